# initial kernel scaffold (unmeasured)
import jax
import jax.numpy as jnp
from jax import lax
from jax.experimental import pallas as pl
from jax.experimental.pallas import tpu as pltpu


def kernel(
    x,
):
    def body(*refs):
        pass

    out_shape = jax.ShapeDtypeStruct(..., jnp.float32)
    return pl.pallas_call(body, out_shape=out_shape)(...)


# baseline (device time: 848244 ns/iter reference)
import functools

import jax
import jax.numpy as jnp
from jax import lax
from jax.experimental import pallas as pl
from jax.experimental.pallas import tpu as pltpu

M = 16384
N = 2048
N_HALF = 1024
CHUNK = 2048
N_CHUNKS = M // CHUNK


def kernel(x):
    def body(x_ref, out_ref, recv_ref, a_ref, b_ref, o_ref,
             copy_sems, send_sem, recv_sem):
        my_x = lax.axis_index("x")
        my_y = lax.axis_index("y")
        my_z = lax.axis_index("z")
        partner = (1 - my_x, my_y, my_z)

        barrier = pltpu.get_barrier_semaphore()
        pl.semaphore_signal(barrier, inc=1, device_id=partner,
                            device_id_type=pl.DeviceIdType.MESH)
        pl.semaphore_wait(barrier, 1)

        partner_lo = (1 - my_x) * N_HALF
        rdma = pltpu.make_async_remote_copy(
            src_ref=x_ref.at[0, :, pl.ds(partner_lo, N_HALF)],
            dst_ref=recv_ref,
            send_sem=send_sem,
            recv_sem=recv_sem,
            device_id=partner,
            device_id_type=pl.DeviceIdType.MESH,
        )
        rdma.start()
        rdma.wait()

        my_lo = my_x * N_HALF

        def chunk_body(i, _):
            r0 = i * CHUNK
            cp_a = pltpu.make_async_copy(
                x_ref.at[0, pl.ds(r0, CHUNK), pl.ds(my_lo, N_HALF)],
                a_ref, copy_sems.at[0])
            cp_b = pltpu.make_async_copy(
                recv_ref.at[pl.ds(r0, CHUNK), :], b_ref, copy_sems.at[1])
            cp_a.start()
            cp_b.start()
            cp_a.wait()
            cp_b.wait()
            o_ref[...] = a_ref[...] + b_ref[...]
            cp_o = pltpu.make_async_copy(
                o_ref, out_ref.at[pl.ds(r0, CHUNK), :], copy_sems.at[2])
            cp_o.start()
            cp_o.wait()
            return 0

        lax.fori_loop(0, N_CHUNKS, chunk_body, 0)

        @functools.partial(pl.run_scoped,
                           exit_sem=pltpu.SemaphoreType.REGULAR)
        def _(exit_sem):
            pl.semaphore_signal(exit_sem, inc=1, device_id=partner,
                                device_id_type=pl.DeviceIdType.MESH)
            pl.semaphore_wait(exit_sem, 1)

    out, _ = pl.pallas_call(
        body,
        out_shape=(
            jax.ShapeDtypeStruct((M, N_HALF), jnp.float32),
            jax.ShapeDtypeStruct((M, N_HALF), jnp.float32),
        ),
        in_specs=[pl.BlockSpec(memory_space=pl.ANY)],
        out_specs=(
            pl.BlockSpec(memory_space=pl.ANY),
            pl.BlockSpec(memory_space=pl.ANY),
        ),
        scratch_shapes=[
            pltpu.VMEM((CHUNK, N_HALF), jnp.float32),
            pltpu.VMEM((CHUNK, N_HALF), jnp.float32),
            pltpu.VMEM((CHUNK, N_HALF), jnp.float32),
            pltpu.SemaphoreType.DMA((3,)),
            pltpu.SemaphoreType.DMA,
            pltpu.SemaphoreType.DMA,
        ],
        compiler_params=pltpu.CompilerParams(collective_id=0),
    )(x)
    return out


# device time: 494152 ns/iter; 1.7166x vs baseline; 1.7166x over previous
import functools

import jax
import jax.numpy as jnp
import numpy as np
from jax import lax
from jax.experimental import pallas as pl
from jax.experimental.pallas import tpu as pltpu

M = 16384
N = 2048
N_HALF = 1024
N_RING = 16
R = M // N_RING
N_FWD = 8
N_BWD = 7

_CYCLE = [
    (0, 0), (0, 1), (0, 2), (0, 3),
    (1, 3), (1, 2), (1, 1),
    (2, 1), (2, 2), (2, 3),
    (3, 3), (3, 2), (3, 1), (3, 0),
    (2, 0), (1, 0),
]
_POS = np.zeros(16, dtype=np.int32)
for _p, (_y, _z) in enumerate(_CYCLE):
    _POS[_y * 4 + _z] = _p
_NEXT_Y = np.array([_CYCLE[(p + 1) % 16][0] for p in range(16)], np.int32)
_NEXT_Z = np.array([_CYCLE[(p + 1) % 16][1] for p in range(16)], np.int32)
_PREV_Y = np.array([_CYCLE[(p - 1) % 16][0] for p in range(16)], np.int32)
_PREV_Z = np.array([_CYCLE[(p - 1) % 16][1] for p in range(16)], np.int32)


def kernel(x):
    my_y = lax.axis_index("y")
    my_z = lax.axis_index("z")
    pos = jnp.asarray(_POS)[my_y * 4 + my_z]
    scalars = jnp.stack(
        [
            pos,
            jnp.asarray(_NEXT_Y)[pos],
            jnp.asarray(_NEXT_Z)[pos],
            jnp.asarray(_PREV_Y)[pos],
            jnp.asarray(_PREV_Z)[pos],
        ]
    ).astype(jnp.int32)

    def body(s_ref, x_ref, out_ref, recv_ref, a_ref, b_ref, o_ref,
             copy_sems, st1_send, st1_recv,
             f_send, f_recv, b_send, b_recv):
        my_x = lax.axis_index("x")
        my_yy = lax.axis_index("y")
        my_zz = lax.axis_index("z")
        pos = s_ref[0]
        nxt = (my_x, s_ref[1], s_ref[2])
        prv = (my_x, s_ref[3], s_ref[4])
        xpartner = (1 - my_x, my_yy, my_zz)

        my_lo = my_x * N_HALF
        partner_lo = (1 - my_x) * N_HALF

        barrier = pltpu.get_barrier_semaphore()
        for nbr in (xpartner, nxt, prv):
            pl.semaphore_signal(barrier, inc=1, device_id=nbr,
                                device_id_type=pl.DeviceIdType.MESH)
        pl.semaphore_wait(barrier, 3)

        def chunk(ref, k):
            return ref.at[pl.ds(k * R, R), :]

        def fwd_rdma(k, s):
            return pltpu.make_async_remote_copy(
                src_ref=chunk(recv_ref, k),
                dst_ref=chunk(recv_ref, k),
                send_sem=f_send.at[s],
                recv_sem=f_recv.at[s],
                device_id=nxt,
                device_id_type=pl.DeviceIdType.MESH,
            )

        def bwd_rdma(k, s):
            return pltpu.make_async_remote_copy(
                src_ref=chunk(recv_ref, k),
                dst_ref=chunk(recv_ref, k),
                send_sem=b_send.at[s],
                recv_sem=b_recv.at[s],
                device_id=prv,
                device_id_type=pl.DeviceIdType.MESH,
            )

        def add_chunk(k):
            cp_a = pltpu.make_async_copy(
                x_ref.at[0, pl.ds(k * R, R), pl.ds(my_lo, N_HALF)],
                a_ref, copy_sems.at[0])
            cp_b = pltpu.make_async_copy(
                chunk(recv_ref, k), b_ref, copy_sems.at[1])
            cp_a.start()
            cp_b.start()
            cp_a.wait()
            cp_b.wait()
            o_ref[...] = a_ref[...] + b_ref[...]
            cp_o = pltpu.make_async_copy(
                o_ref, chunk(out_ref, k), copy_sems.at[2])
            cp_o.start()
            cp_o.wait()

        st1 = pltpu.make_async_remote_copy(
            src_ref=x_ref.at[0, pl.ds(pos * R, R), pl.ds(partner_lo, N_HALF)],
            dst_ref=chunk(recv_ref, pos),
            send_sem=st1_send,
            recv_sem=st1_recv,
            device_id=xpartner,
            device_id_type=pl.DeviceIdType.MESH,
        )
        st1.start()
        st1.wait()

        fwd_rdma(pos, 0).start()
        bwd_rdma(pos, 0).start()
        add_chunk(pos)

        for s in range(N_FWD):
            kf = (pos + N_RING - 1 - s) % N_RING
            fwd_rdma(kf, s).wait_recv()
            if s + 1 < N_FWD:
                fwd_rdma(kf, s + 1).start()
            if s < N_BWD:
                kb = (pos + 1 + s) % N_RING
                bwd_rdma(kb, s).wait_recv()
                if s + 1 < N_BWD:
                    bwd_rdma(kb, s + 1).start()
            add_chunk(kf)
            if s < N_BWD:
                add_chunk(kb)

        for s in range(N_FWD):
            fwd_rdma(pos, s).wait_send()
        for s in range(N_BWD):
            bwd_rdma(pos, s).wait_send()

        @functools.partial(pl.run_scoped,
                           exit_sem=pltpu.SemaphoreType.REGULAR)
        def _(exit_sem):
            for nbr in (xpartner, nxt, prv):
                pl.semaphore_signal(exit_sem, inc=1, device_id=nbr,
                                    device_id_type=pl.DeviceIdType.MESH)
            pl.semaphore_wait(exit_sem, 3)

    out, _ = pl.pallas_call(
        body,
        out_shape=(
            jax.ShapeDtypeStruct((M, N_HALF), jnp.float32),
            jax.ShapeDtypeStruct((M, N_HALF), jnp.float32),
        ),
        in_specs=[
            pl.BlockSpec(memory_space=pltpu.MemorySpace.SMEM),
            pl.BlockSpec(memory_space=pl.ANY),
        ],
        out_specs=(
            pl.BlockSpec(memory_space=pl.ANY),
            pl.BlockSpec(memory_space=pl.ANY),
        ),
        scratch_shapes=[
            pltpu.VMEM((R, N_HALF), jnp.float32),
            pltpu.VMEM((R, N_HALF), jnp.float32),
            pltpu.VMEM((R, N_HALF), jnp.float32),
            pltpu.SemaphoreType.DMA((3,)),
            pltpu.SemaphoreType.DMA,
            pltpu.SemaphoreType.DMA,
            pltpu.SemaphoreType.DMA((N_FWD,)),
            pltpu.SemaphoreType.DMA((N_FWD,)),
            pltpu.SemaphoreType.DMA((N_BWD,)),
            pltpu.SemaphoreType.DMA((N_BWD,)),
        ],
        compiler_params=pltpu.CompilerParams(collective_id=0),
    )(scalars, x)
    return out
